# baseline (speedup 1.0000x reference)
import numpy as np
import jax
import jax.numpy as jnp
from jax.experimental import pallas as pl
from jax.experimental.pallas import tpu as pltpu

_ROWS = 400
_CHUNK = 80
_CW = 1664
_NCB = 6
_TAIL = 16
_NBUF = 3


def _body(c1m_ref, r2_ref, cb2_ref,
          x_ref, w1t_ref, b1_ref, w2t_ref, b2_ref, c1_ref, c2_ref,
          o_ref, xp_ref, hp_ref, tail_ref, buf_ref, sem_ref):
    t = pl.program_id(0)
    steps = pl.num_programs(0)
    nb = 10000 // _ROWS

    def start_tile(p, slot):
        r = r2_ref[nb + p]
        cb = cb2_ref[nb + p]
        pltpu.make_async_copy(
            c2_ref.at[pl.ds(r * _ROWS, _ROWS),
                      pl.ds(pl.multiple_of(cb * _CW, 128), _CW)],
            buf_ref.at[slot],
            sem_ref.at[slot],
        ).start()

    @pl.when(t == 0)
    def _():
        xp = jnp.dot(x_ref[...].astype(jnp.bfloat16), w1t_ref[...],
                     preferred_element_type=jnp.float32)
        xp_ref[...] = xp.astype(jnp.bfloat16)

    @pl.when(t < nb)
    def _():
        r = t
        o_ref[pl.ds(r * _ROWS, _ROWS), :] = jnp.broadcast_to(
            b2_ref[...], (_ROWS, b2_ref.shape[1]))
        acc = None
        for cb in range(_NCB):
            cbf = c1_ref[:, pl.ds(cb * _CW, _CW)].astype(jnp.bfloat16)
            part = jax.lax.dot_general(
                cbf, xp_ref[pl.ds(cb * _CW, _CW), :],
                (((1,), (0,)), ((), ())),
                preferred_element_type=jnp.float32)
            acc = part if acc is None else acc + part

            @pl.when(_CW * (cb + 1) <= _ROWS * t)
            def _(cbf=cbf, cb=cb):
                contrib = jax.lax.dot_general(
                    cbf, hp_ref[pl.ds(cb * _CW, _CW), :],
                    (((1,), (0,)), ((), ())),
                    preferred_element_type=jnp.float32)
                o_ref[pl.ds(r * _ROWS, _ROWS), :] = (
                    o_ref[pl.ds(r * _ROWS, _ROWS), :] + contrib)
        tailbf = c1_ref[:, pl.ds(_NCB * _CW, _TAIL)].astype(jnp.bfloat16)
        tail_ref[pl.ds(r * _ROWS, _ROWS), :] = tailbf
        acc = acc + jax.lax.dot_general(
            tailbf, xp_ref[pl.ds(_NCB * _CW, _TAIL), :],
            (((1,), (0,)), ((), ())),
            preferred_element_type=jnp.float32)
        h = jnp.maximum(acc + b1_ref[...], 0.0)
        hp = jnp.dot(h.astype(jnp.bfloat16), w2t_ref[...],
                     preferred_element_type=jnp.float32)
        hp_ref[pl.ds(r * _ROWS, _ROWS), :] = hp.astype(jnp.bfloat16)

    @pl.when(t == nb - 1)
    def _():
        for p in range(_NBUF - 1):
            start_tile(p, p)

    @pl.when(t >= nb)
    def _():
        p = t - nb
        slot = jax.lax.rem(p, _NBUF)

        @pl.when(t + _NBUF - 1 < steps)
        def _():
            start_tile(p + _NBUF - 1, jax.lax.rem(p + _NBUF - 1, _NBUF))

        pltpu.make_async_copy(
            buf_ref.at[slot], buf_ref.at[slot], sem_ref.at[slot]).wait()
        r = r2_ref[t]
        cb = cb2_ref[t]
        c = buf_ref[slot].astype(jnp.bfloat16)
        contrib = jax.lax.dot_general(
            c, hp_ref[pl.ds(cb * _CW, _CW), :],
            (((1,), (0,)), ((), ())),
            preferred_element_type=jnp.float32)
        o_ref[pl.ds(r * _ROWS, _ROWS), :] = (
            o_ref[pl.ds(r * _ROWS, _ROWS), :] + contrib)

    @pl.when(t == steps - 1)
    def _():
        o_ref[...] = o_ref[...] + jax.lax.dot_general(
            tail_ref[...], hp_ref[pl.ds(_NCB * _CW, _TAIL), :],
            (((1,), (0,)), ((), ())),
            preferred_element_type=jnp.float32)


def kernel(x, edge_index, C, W1, b1, W2, b2):
    del edge_index
    n, in_dim = x.shape
    hid = W1.shape[0]
    ncls = W2.shape[0]
    nb = n // _ROWS

    p2 = [(r, cb) for r in range(nb)
          for cb in range(min(_NCB, (_ROWS * r) // _CW), _NCB)]
    steps = nb + len(p2)
    c1m = np.minimum(np.arange(steps), nb - 1).astype(np.int32)
    r2 = np.zeros(steps, np.int32)
    cb2 = np.zeros(steps, np.int32)
    for i, (r, cb) in enumerate(p2):
        r2[nb + i] = r
        cb2[nb + i] = cb

    grid_spec = pltpu.PrefetchScalarGridSpec(
        num_scalar_prefetch=3,
        grid=(steps,),
        in_specs=[
            pl.BlockSpec((n, in_dim), lambda i, a, b, c: (0, 0)),
            pl.BlockSpec((in_dim, hid), lambda i, a, b, c: (0, 0)),
            pl.BlockSpec((1, hid), lambda i, a, b, c: (0, 0)),
            pl.BlockSpec((hid, ncls), lambda i, a, b, c: (0, 0)),
            pl.BlockSpec((1, ncls), lambda i, a, b, c: (0, 0)),
            pl.BlockSpec((_ROWS, n), lambda i, a, b, c: (a[i], 0)),
            pl.BlockSpec(memory_space=pl.ANY),
        ],
        out_specs=pl.BlockSpec((n, ncls), lambda i, a, b, c: (0, 0)),
        scratch_shapes=[
            pltpu.VMEM((n, hid), jnp.bfloat16),
            pltpu.VMEM((n, ncls), jnp.bfloat16),
            pltpu.VMEM((n, _TAIL), jnp.bfloat16),
            pltpu.VMEM((_NBUF, _ROWS, _CW), jnp.float32),
            pltpu.SemaphoreType.DMA((_NBUF,)),
        ],
    )

    return pl.pallas_call(
        _body,
        grid_spec=grid_spec,
        out_shape=jax.ShapeDtypeStruct((n, ncls), jnp.float32),
    )(c1m, r2, cb2,
      x, W1.T.astype(jnp.bfloat16), b1.reshape(1, hid),
      W2.T.astype(jnp.bfloat16), b2.reshape(1, ncls), C, C)


# speedup vs baseline: 1.0908x; 1.0908x over previous
import numpy as np
import jax
import jax.numpy as jnp
from jax.experimental import pallas as pl
from jax.experimental.pallas import tpu as pltpu

_ROWS = 400
_CHUNK = 80
_CW = 1664
_NCB = 6
_TAIL = 16
_NBUF = 4


def _body(c1m_ref, r2_ref, cb2_ref,
          x_ref, w1t_ref, b1_ref, w2t_ref, b2_ref, c1_ref, c2_ref,
          o_ref, xp_ref, hp_ref, tail_ref, buf_ref, sem_ref):
    t = pl.program_id(0)
    steps = pl.num_programs(0)
    nb = 10000 // _ROWS

    def start_tile(p, slot):
        r = r2_ref[nb + p]
        cb = cb2_ref[nb + p]
        pltpu.make_async_copy(
            c2_ref.at[pl.ds(r * _ROWS, _ROWS),
                      pl.ds(pl.multiple_of(cb * _CW, 128), _CW)],
            buf_ref.at[slot],
            sem_ref.at[slot],
        ).start()

    @pl.when(t == 0)
    def _():
        xp = jnp.dot(x_ref[...].astype(jnp.bfloat16), w1t_ref[...],
                     preferred_element_type=jnp.float32)
        xp_ref[...] = xp.astype(jnp.bfloat16)

    @pl.when(t < nb)
    def _():
        r = t
        xp = xp_ref[...]
        for k in range(_ROWS // _CHUNK):
            c = c1_ref[pl.ds(k * _CHUNK, _CHUNK), :].astype(jnp.bfloat16)
            h = jax.lax.dot_general(
                c, xp, (((1,), (0,)), ((), ())),
                preferred_element_type=jnp.float32)
            h = jnp.maximum(h + b1_ref[...], 0.0)
            hp = jnp.dot(h.astype(jnp.bfloat16), w2t_ref[...],
                         preferred_element_type=jnp.float32)
            hp_ref[pl.ds(r * _ROWS + k * _CHUNK, _CHUNK), :] = (
                hp.astype(jnp.bfloat16))
        tail_ref[pl.ds(r * _ROWS, _ROWS), :] = (
            c1_ref[:, pl.ds(_NCB * _CW, _TAIL)].astype(jnp.bfloat16))
        acc = jnp.broadcast_to(b2_ref[...], (_ROWS, b2_ref.shape[1]))
        o_ref[pl.ds(r * _ROWS, _ROWS), :] = acc
        for cb in range(_NCB):
            @pl.when(_CW * (cb + 1) <= _ROWS * (t + 1))
            def _():
                c = c1_ref[:, pl.ds(cb * _CW, _CW)].astype(jnp.bfloat16)
                contrib = jax.lax.dot_general(
                    c, hp_ref[pl.ds(cb * _CW, _CW), :],
                    (((1,), (0,)), ((), ())),
                    preferred_element_type=jnp.float32)
                o_ref[pl.ds(r * _ROWS, _ROWS), :] = (
                    o_ref[pl.ds(r * _ROWS, _ROWS), :] + contrib)

    @pl.when(t == nb - 1)
    def _():
        for p in range(_NBUF - 1):
            start_tile(p, p)

    @pl.when(t >= nb)
    def _():
        p = t - nb
        slot = jax.lax.rem(p, _NBUF)

        @pl.when(t + _NBUF - 1 < steps)
        def _():
            start_tile(p + _NBUF - 1, jax.lax.rem(p + _NBUF - 1, _NBUF))

        pltpu.make_async_copy(
            buf_ref.at[slot], buf_ref.at[slot], sem_ref.at[slot]).wait()
        r = r2_ref[t]
        cb = cb2_ref[t]
        c = buf_ref[slot].astype(jnp.bfloat16)
        contrib = jax.lax.dot_general(
            c, hp_ref[pl.ds(cb * _CW, _CW), :],
            (((1,), (0,)), ((), ())),
            preferred_element_type=jnp.float32)
        o_ref[pl.ds(r * _ROWS, _ROWS), :] = (
            o_ref[pl.ds(r * _ROWS, _ROWS), :] + contrib)

    @pl.when(t == steps - 1)
    def _():
        o_ref[...] = o_ref[...] + jax.lax.dot_general(
            tail_ref[...], hp_ref[pl.ds(_NCB * _CW, _TAIL), :],
            (((1,), (0,)), ((), ())),
            preferred_element_type=jnp.float32)


def kernel(x, edge_index, C, W1, b1, W2, b2):
    del edge_index
    n, in_dim = x.shape
    hid = W1.shape[0]
    ncls = W2.shape[0]
    nb = n // _ROWS

    p2 = [(r, cb) for r in range(nb)
          for cb in range(min(_NCB, (_ROWS * (r + 1)) // _CW), _NCB)]
    steps = nb + len(p2)
    c1m = np.minimum(np.arange(steps), nb - 1).astype(np.int32)
    r2 = np.zeros(steps, np.int32)
    cb2 = np.zeros(steps, np.int32)
    for i, (r, cb) in enumerate(p2):
        r2[nb + i] = r
        cb2[nb + i] = cb

    grid_spec = pltpu.PrefetchScalarGridSpec(
        num_scalar_prefetch=3,
        grid=(steps,),
        in_specs=[
            pl.BlockSpec((n, in_dim), lambda i, a, b, c: (0, 0)),
            pl.BlockSpec((in_dim, hid), lambda i, a, b, c: (0, 0)),
            pl.BlockSpec((1, hid), lambda i, a, b, c: (0, 0)),
            pl.BlockSpec((hid, ncls), lambda i, a, b, c: (0, 0)),
            pl.BlockSpec((1, ncls), lambda i, a, b, c: (0, 0)),
            pl.BlockSpec((_ROWS, n), lambda i, a, b, c: (a[i], 0)),
            pl.BlockSpec(memory_space=pl.ANY),
        ],
        out_specs=pl.BlockSpec((n, ncls), lambda i, a, b, c: (0, 0)),
        scratch_shapes=[
            pltpu.VMEM((n, hid), jnp.bfloat16),
            pltpu.VMEM((n, ncls), jnp.bfloat16),
            pltpu.VMEM((n, _TAIL), jnp.bfloat16),
            pltpu.VMEM((_NBUF, _ROWS, _CW), jnp.float32),
            pltpu.SemaphoreType.DMA((_NBUF,)),
        ],
    )

    return pl.pallas_call(
        _body,
        grid_spec=grid_spec,
        out_shape=jax.ShapeDtypeStruct((n, ncls), jnp.float32),
    )(c1m, r2, cb2,
      x, W1.T.astype(jnp.bfloat16), b1.reshape(1, hid),
      W2.T.astype(jnp.bfloat16), b2.reshape(1, ncls), C, C)
